# Initial kernel scaffold; baseline (speedup 1.0000x reference)
import jax
import jax.numpy as jnp
from jax.experimental import pallas as pl


def kernel(input_ids, tables):
    raise NotImplementedError("write your pallas kernel here")


# speedup vs baseline: 2.3315x; 2.3315x over previous
import functools

import jax
import jax.numpy as jnp
from jax import lax
from jax.experimental import pallas as pl
from jax.experimental.pallas import tpu as pltpu
from jax.experimental.pallas import tpu_sc as plsc

NC, NS, L = 2, 16, 16
NW = NC * NS

K = 64
V = 513
D = 768
R = 512
RPW = R // NW
IPW = RPW * K

_mesh = plsc.VectorSubcoreMesh(core_axis_name="c", subcore_axis_name="s")


@functools.partial(
    pl.kernel,
    out_type=jax.ShapeDtypeStruct((R, D), jnp.float32),
    mesh=_mesh,
    scratch_types=[
        pltpu.VMEM((IPW,), jnp.int32),
        pltpu.VMEM((K, D), jnp.float32),
        pltpu.VMEM((RPW, D), jnp.float32),
        pltpu.SemaphoreType.DMA,
    ],
)
def _emb_sum(ids_hbm, table_hbm, out_hbm, idx_v, rows_v, acc_v, sem):
    wid = lax.axis_index("s") * NC + lax.axis_index("c")
    base = wid * IPW
    pltpu.sync_copy(ids_hbm.at[pl.ds(base, IPW)], idx_v)

    lane = lax.iota(jnp.int32, L)

    def fix(c, _):
        off = lane * V + (V * L) * lax.rem(c, K // L)
        idx_v[pl.ds(c * L, L)] = idx_v[pl.ds(c * L, L)] + off
        return 0

    lax.fori_loop(0, IPW // L, fix, 0)

    def per_row(j, _):
        pltpu.async_copy(table_hbm.at[idx_v.at[pl.ds(j * K, K)]], rows_v,
                         sem).wait()
        for dc in range(D // (8 * L)):
            col0 = dc * 8 * L
            init = tuple(rows_v[0, pl.ds(col0 + t * L, L)] for t in range(8))

            def body(i, carry):
                return tuple(carry[t] + rows_v[i, pl.ds(col0 + t * L, L)]
                             for t in range(8))

            accs = lax.fori_loop(1, K, body, init)
            for t in range(8):
                acc_v[j, pl.ds(col0 + t * L, L)] = accs[t]
        return 0

    lax.fori_loop(0, RPW, per_row, 0)
    pltpu.sync_copy(acc_v, out_hbm.at[pl.ds(wid * RPW, RPW)])


def kernel(input_ids, tables):
    b, seq = input_ids.shape
    ids = input_ids.astype(jnp.int32).reshape(-1)
    flat_tables = tables.reshape(K * V, D)
    out = _emb_sum(ids, flat_tables)
    return out.reshape(b, seq // K, D)


# speedup vs baseline: 2.5454x; 1.0917x over previous
import functools

import jax
import jax.numpy as jnp
from jax import lax
from jax.experimental import pallas as pl
from jax.experimental.pallas import tpu as pltpu
from jax.experimental.pallas import tpu_sc as plsc

NC, NS, L = 2, 16, 16
NW = NC * NS

K = 64
V = 513
D = 768
R = 512
RPW = R // NW
IPW = RPW * K

_mesh = plsc.VectorSubcoreMesh(core_axis_name="c", subcore_axis_name="s")


@functools.partial(
    pl.kernel,
    out_type=jax.ShapeDtypeStruct((R, D), jnp.float32),
    mesh=_mesh,
    scratch_types=[
        pltpu.VMEM((IPW,), jnp.int32),
        pltpu.VMEM((K, D), jnp.float32),
        pltpu.VMEM((K, D), jnp.float32),
        pltpu.VMEM((RPW, D), jnp.float32),
        pltpu.SemaphoreType.DMA,
        pltpu.SemaphoreType.DMA,
    ],
)
def _emb_sum(ids_hbm, table_hbm, out_hbm, idx_v, rows_a, rows_b, acc_v,
             sem_a, sem_b):
    wid = lax.axis_index("s") * NC + lax.axis_index("c")
    base = wid * IPW
    pltpu.sync_copy(ids_hbm.at[pl.ds(base, IPW)], idx_v)

    lane = lax.iota(jnp.int32, L)

    def fix(c, _):
        off = lane * V + (V * L) * lax.rem(c, K // L)
        idx_v[pl.ds(c * L, L)] = idx_v[pl.ds(c * L, L)] + off
        return 0

    lax.fori_loop(0, IPW // L, fix, 0)

    def gather(j, buf, sem):
        return pltpu.make_async_copy(
            table_hbm.at[idx_v.at[pl.ds(j * K, K)]], buf, sem)

    def accum(buf, j):
        for dc in range(D // (8 * L)):
            col0 = dc * 8 * L
            init = tuple(buf[0, pl.ds(col0 + t * L, L)] for t in range(8))

            def body(i, carry):
                return tuple(carry[t] + buf[i, pl.ds(col0 + t * L, L)]
                             for t in range(8))

            accs = lax.fori_loop(1, K, body, init, unroll=8)
            for t in range(8):
                acc_v[j, pl.ds(col0 + t * L, L)] = accs[t]

    gather(0, rows_a, sem_a).start()

    def pair(h, _):
        j0 = 2 * h
        gather(j0 + 1, rows_b, sem_b).start()
        gather(j0, rows_a, sem_a).wait()
        accum(rows_a, j0)

        @pl.when(j0 + 2 < RPW)
        def _():
            gather(j0 + 2, rows_a, sem_a).start()

        gather(j0 + 1, rows_b, sem_b).wait()
        accum(rows_b, j0 + 1)
        return 0

    lax.fori_loop(0, RPW // 2, pair, 0)
    pltpu.sync_copy(acc_v, out_hbm.at[pl.ds(wid * RPW, RPW)])


def kernel(input_ids, tables):
    b, seq = input_ids.shape
    ids = input_ids.astype(jnp.int32).reshape(-1)
    flat_tables = tables.reshape(K * V, D)
    out = _emb_sum(ids, flat_tables)
    return out.reshape(b, seq // K, D)


# speedup vs baseline: 2.5482x; 1.0011x over previous
import functools

import jax
import jax.numpy as jnp
from jax import lax
from jax.experimental import pallas as pl
from jax.experimental.pallas import tpu as pltpu
from jax.experimental.pallas import tpu_sc as plsc

NC, NS, L = 2, 16, 16
NW = NC * NS

K = 64
V = 513
D = 768
R = 512
RPW = R // NW
IPW = RPW * K

_mesh = plsc.VectorSubcoreMesh(core_axis_name="c", subcore_axis_name="s")


@functools.partial(
    pl.kernel,
    out_type=jax.ShapeDtypeStruct((R, D), jnp.float32),
    mesh=_mesh,
    scratch_types=[
        pltpu.VMEM((IPW,), jnp.int32),
        pltpu.VMEM((K, D), jnp.float32),
        pltpu.VMEM((K, D), jnp.float32),
        pltpu.VMEM((RPW, D), jnp.float32),
        pltpu.SemaphoreType.DMA,
        pltpu.SemaphoreType.DMA,
    ],
    compiler_params=pltpu.CompilerParams(use_tc_tiling_on_sc=True),
)
def _emb_sum(ids_hbm, table_hbm, out_hbm, idx_v, rows_a, rows_b, acc_v,
             sem_a, sem_b):
    wid = lax.axis_index("s") * NC + lax.axis_index("c")
    base = wid * IPW
    pltpu.sync_copy(ids_hbm.at[pl.ds(base, IPW)], idx_v)

    lane = lax.iota(jnp.int32, L)

    def fix(c, _):
        off = lane * V + (V * L) * lax.rem(c, K // L)
        idx_v[pl.ds(c * L, L)] = idx_v[pl.ds(c * L, L)] + off
        return 0

    lax.fori_loop(0, IPW // L, fix, 0)

    def gather(j, buf, sem):
        return pltpu.make_async_copy(
            table_hbm.at[idx_v.at[pl.ds(j * K, K)]], buf, sem)

    def accum(buf, j):
        for dc in range(D // (8 * L)):
            col0 = dc * 8 * L
            init = tuple(buf[0, pl.ds(col0 + t * L, L)] for t in range(8))

            def body(i, carry):
                return tuple(carry[t] + buf[i, pl.ds(col0 + t * L, L)]
                             for t in range(8))

            accs = lax.fori_loop(1, K, body, init, unroll=8)
            for t in range(8):
                acc_v[j, pl.ds(col0 + t * L, L)] = accs[t]

    gather(0, rows_a, sem_a).start()

    def pair(h, _):
        j0 = 2 * h
        gather(j0 + 1, rows_b, sem_b).start()
        gather(j0, rows_a, sem_a).wait()
        accum(rows_a, j0)

        @pl.when(j0 + 2 < RPW)
        def _():
            gather(j0 + 2, rows_a, sem_a).start()

        gather(j0 + 1, rows_b, sem_b).wait()
        accum(rows_b, j0 + 1)
        return 0

    lax.fori_loop(0, RPW // 2, pair, 0)
    pltpu.sync_copy(acc_v, out_hbm.at[pl.ds(wid * RPW, RPW)])


def kernel(input_ids, tables):
    b, seq = input_ids.shape
    ids = input_ids.astype(jnp.int32).reshape(-1)
    flat_tables = tables.reshape(K * V, D)
    out = _emb_sum(ids, flat_tables)
    return out.reshape(b, seq // K, D)
